# baseline (speedup 1.0000x reference)
import functools

import jax
import jax.numpy as jnp
from jax import lax
from jax.experimental import pallas as pl
from jax.experimental.pallas import tpu as pltpu
from jax.experimental.pallas import tpu_sc as plsc

N_HEADS = 4
OUT_F = 32
HF = N_HEADS * OUT_F
DP = 8
NC, NS = 2, 16
NW = NC * NS
C = 80
G = 5
LANES = 16


def _tc_prep_body(x_ref, wt_ref, aa_ref, wh_ref, al_ref):
    wh = jnp.dot(x_ref[...], wt_ref[...], preferred_element_type=jnp.float32)
    wh_ref[...] = wh
    al_ref[...] = jnp.dot(wh, aa_ref[...], preferred_element_type=jnp.float32)


def _tc_finish_body(pn_ref, pd_ref, s_ref, o_ref):
    tot = pn_ref[0] + pn_ref[1]
    den = pd_ref[0, :, :N_HEADS] + pd_ref[1, :, :N_HEADS]
    mult = jnp.dot(1.0 / (den + 1e-10), s_ref[...],
                   preferred_element_type=jnp.float32)
    o_ref[...] = tot * mult


def _sc_agg_body(wh_hbm, al_hbm, src3_hbm, dst3_hbm, onum_hbm, oden_hbm,
                 rows2, asrc2, adst2, dbuf2, sgidx, dgidx,
                 numer_s, denom_s, gsem, isem):
    n = wh_hbm.shape[0]
    ngrp = src3_hbm.shape[1]
    nchunk = ngrp * G
    c = lax.axis_index("c")
    s = lax.axis_index("s")
    w = c * NS + s

    rpt = n // NS
    row0 = s * rpt
    nfull = rpt // C
    rem = rpt - nfull * C

    zeros16 = jnp.zeros((LANES,), jnp.float32)

    def _zrows(i, _):
        r = i // (HF // LANES)
        j = i % (HF // LANES)
        rows2[0, r, pl.ds(j * LANES, LANES)] = zeros16
        return 0

    lax.fori_loop(0, C * (HF // LANES), _zrows, 0)

    lanes0 = lax.iota(jnp.int32, LANES)
    zrow = lax.shift_right_logical(lanes0, 3)
    zcol = lanes0 & 7

    for b in range(2):
        def _zdbuf(i, _):
            plsc.store_scatter(dbuf2.at[b], [zrow + i * 2, zcol], zeros16)
            return 0

        lax.fori_loop(0, C * DP // LANES, _zdbuf, 0)

    def _zcopy(j, _):
        pltpu.sync_copy(rows2.at[0], numer_s.at[pl.ds(row0 + j * C, C)])
        pltpu.sync_copy(dbuf2.at[0], denom_s.at[pl.ds(row0 + j * C, C)])
        return 0

    lax.fori_loop(0, nfull, _zcopy, 0)
    pltpu.sync_copy(rows2.at[0, pl.ds(0, rem)],
                    numer_s.at[pl.ds(row0 + nfull * C, rem)])
    pltpu.sync_copy(dbuf2.at[0, pl.ds(0, rem)],
                    denom_s.at[pl.ds(row0 + nfull * C, rem)])

    plsc.subcore_barrier()

    lanes = lax.iota(jnp.int32, LANES)

    def _sidx(t):
        g = t // G
        p = t - g * G
        return sgidx.at[g % 2, pl.ds(p * C, C)], dgidx.at[g % 2, pl.ds(p * C, C)]

    def _start_gathers(t):
        par = t % 2
        ssl, dsl = _sidx(t)
        pltpu.make_async_copy(wh_hbm.at[ssl], rows2.at[par], gsem.at[par]).start()
        pltpu.make_async_copy(al_hbm.at[ssl], asrc2.at[par], gsem.at[par]).start()
        pltpu.make_async_copy(al_hbm.at[dsl], adst2.at[par], gsem.at[par]).start()

    def _wait_gathers(t):
        par = t % 2
        ssl, dsl = _sidx(t)
        pltpu.make_async_copy(wh_hbm.at[ssl], rows2.at[par], gsem.at[par]).wait()
        pltpu.make_async_copy(al_hbm.at[ssl], asrc2.at[par], gsem.at[par]).wait()
        pltpu.make_async_copy(al_hbm.at[dsl], adst2.at[par], gsem.at[par]).wait()

    def _start_idx_group(q):
        qb = q % 2
        pltpu.make_async_copy(src3_hbm.at[w, q], sgidx.at[qb], isem.at[qb]).start()
        pltpu.make_async_copy(dst3_hbm.at[w, q], dgidx.at[qb], isem.at[qb]).start()

    def _wait_idx_group(q):
        qb = q % 2
        pltpu.make_async_copy(src3_hbm.at[w, q], sgidx.at[qb], isem.at[qb]).wait()
        pltpu.make_async_copy(dst3_hbm.at[w, q], dgidx.at[qb], isem.at[qb]).wait()

    pltpu.sync_copy(src3_hbm.at[w, 0], sgidx.at[0])
    pltpu.sync_copy(dst3_hbm.at[w, 0], dgidx.at[0])
    _start_idx_group(1)
    _start_gathers(0)

    def _chunk_body(t, _):
        par = t % 2
        _wait_gathers(t)

        rows_r = rows2.at[par]
        asrc_r = asrc2.at[par]
        adst_r = adst2.at[par]
        dbuf_r = dbuf2.at[par]

        def _group_body(g, _):
            erow = g * LANES + lanes
            exs = []
            for h in range(N_HEADS):
                ch = jnp.full((LANES,), h, jnp.int32)
                ea = plsc.load_gather(asrc_r, [erow, ch])
                eb = plsc.load_gather(adst_r, [erow, ch + N_HEADS])
                ev = ea + eb
                ev = jnp.where(ev > 0, ev, ev * 0.2)
                ex = jnp.exp(ev)
                plsc.store_scatter(dbuf_r, [erow, ch], ex)
                exs.append(ex)
            for h in range(N_HEADS):
                for f in range(OUT_F):
                    col = jnp.full((LANES,), h * OUT_F + f, jnp.int32)
                    v = plsc.load_gather(rows_r, [erow, col])
                    plsc.store_scatter(rows_r, [erow, col], v * exs[h])
            return 0

        lax.fori_loop(0, C // LANES, _group_body, 0)

        t1 = t + 1

        @pl.when(t1 < nchunk)
        def _():
            @pl.when(t1 % G == 0)
            def _():
                _wait_idx_group(t1 // G)
            _start_gathers(t1)

        _, dsl = _sidx(t)
        pltpu.sync_copy(rows_r, numer_s.at[dsl], add=True)
        pltpu.sync_copy(dbuf_r, denom_s.at[dsl], add=True)

        @pl.when((t1 < nchunk) & (t1 % G == 0))
        def _():
            q1 = t1 // G + 1

            @pl.when(q1 < ngrp)
            def _():
                _start_idx_group(q1)

        return 0

    lax.fori_loop(0, nchunk, _chunk_body, 0)

    plsc.subcore_barrier()

    def _ocopy(j, _):
        r0 = row0 + j * C
        pltpu.sync_copy(numer_s.at[pl.ds(r0, C)], rows2.at[0])
        pltpu.sync_copy(rows2.at[0], onum_hbm.at[c, pl.ds(r0, C)])
        pltpu.sync_copy(denom_s.at[pl.ds(r0, C)], dbuf2.at[0])
        pltpu.sync_copy(dbuf2.at[0], oden_hbm.at[c, pl.ds(r0, C)])
        return 0

    lax.fori_loop(0, nfull, _ocopy, 0)
    r0 = row0 + nfull * C
    pltpu.sync_copy(numer_s.at[pl.ds(r0, rem)], rows2.at[0, pl.ds(0, rem)])
    pltpu.sync_copy(rows2.at[0, pl.ds(0, rem)], onum_hbm.at[c, pl.ds(r0, rem)])
    pltpu.sync_copy(denom_s.at[pl.ds(r0, rem)], dbuf2.at[0, pl.ds(0, rem)])
    pltpu.sync_copy(dbuf2.at[0, pl.ds(0, rem)], oden_hbm.at[c, pl.ds(r0, rem)])


def kernel(x, edge_index, W, a):
    n, in_f = x.shape
    e = edge_index.shape[1]
    epw = e // NW
    ngrp = epw // (G * C)
    src3 = edge_index[0].reshape(NW, ngrp, G * C)
    dst3 = edge_index[1].reshape(NW, ngrp, G * C)
    wt = W.T

    onehot = jnp.eye(N_HEADS, dtype=x.dtype)
    a_src = a[:, :OUT_F]
    a_dst = a[:, OUT_F:]
    aa = jnp.concatenate(
        [a_src[:, :, None] * onehot[:, None, :],
         a_dst[:, :, None] * onehot[:, None, :]], axis=-1).reshape(HF, 2 * N_HEADS)

    sel = jnp.kron(jnp.eye(N_HEADS, dtype=x.dtype), jnp.ones((1, OUT_F), x.dtype))

    bn = 1000
    wh, al = pl.pallas_call(
        _tc_prep_body,
        grid=(n // bn,),
        in_specs=[
            pl.BlockSpec((bn, in_f), lambda i: (i, 0)),
            pl.BlockSpec((in_f, HF), lambda i: (0, 0)),
            pl.BlockSpec((HF, 2 * N_HEADS), lambda i: (0, 0)),
        ],
        out_specs=[
            pl.BlockSpec((bn, HF), lambda i: (i, 0)),
            pl.BlockSpec((bn, 2 * N_HEADS), lambda i: (i, 0)),
        ],
        out_shape=[
            jax.ShapeDtypeStruct((n, HF), jnp.float32),
            jax.ShapeDtypeStruct((n, 2 * N_HEADS), jnp.float32),
        ],
    )(x, wt, aa)

    mesh = plsc.VectorSubcoreMesh(core_axis_name="c", subcore_axis_name="s",
                                  num_cores=NC, num_subcores=NS)
    sc_agg = pl.kernel(
        _sc_agg_body,
        out_type=[
            jax.ShapeDtypeStruct((NC, n, HF), jnp.float32),
            jax.ShapeDtypeStruct((NC, n, DP), jnp.float32),
        ],
        mesh=mesh,
        compiler_params=pltpu.CompilerParams(use_tc_tiling_on_sc=False,
                                             needs_layout_passes=False),
        scratch_types=[
            pltpu.VMEM((2, C, HF), jnp.float32),
            pltpu.VMEM((2, C, 2 * N_HEADS), jnp.float32),
            pltpu.VMEM((2, C, 2 * N_HEADS), jnp.float32),
            pltpu.VMEM((2, C, DP), jnp.float32),
            pltpu.VMEM((2, G * C), jnp.int32),
            pltpu.VMEM((2, G * C), jnp.int32),
            pltpu.VMEM_SHARED((n, HF), jnp.float32),
            pltpu.VMEM_SHARED((n, DP), jnp.float32),
            pltpu.SemaphoreType.DMA((2,)),
            pltpu.SemaphoreType.DMA((2,)),
        ],
    )
    pnum, pden = sc_agg(wh, al, src3, dst3)

    out = pl.pallas_call(
        _tc_finish_body,
        grid=(n // bn,),
        in_specs=[
            pl.BlockSpec((NC, bn, HF), lambda i: (0, i, 0)),
            pl.BlockSpec((NC, bn, DP), lambda i: (0, i, 0)),
            pl.BlockSpec((N_HEADS, HF), lambda i: (0, 0)),
        ],
        out_specs=pl.BlockSpec((bn, HF), lambda i: (i, 0)),
        out_shape=jax.ShapeDtypeStruct((n, HF), jnp.float32),
    )(pnum, pden, sel)
    return out


# speedup vs baseline: 1.0883x; 1.0883x over previous
import functools

import jax
import jax.numpy as jnp
from jax import lax
from jax.experimental import pallas as pl
from jax.experimental.pallas import tpu as pltpu
from jax.experimental.pallas import tpu_sc as plsc

N_HEADS = 4
OUT_F = 32
HF = N_HEADS * OUT_F
DP = 8
NC, NS = 2, 16
NW = NC * NS
C = 80
G = 5
LANES = 16


def _tc_prep_body(x_ref, wt_ref, aa_ref, wh_ref, al_ref):
    wh = jnp.dot(x_ref[...], wt_ref[...], preferred_element_type=jnp.float32)
    wh_ref[...] = wh
    al_ref[...] = jnp.dot(wh, aa_ref[...], preferred_element_type=jnp.float32)


def _tc_finish_body(pn_ref, pd_ref, s_ref, o_ref):
    tot = pn_ref[0] + pn_ref[1]
    den = pd_ref[0, :, :N_HEADS] + pd_ref[1, :, :N_HEADS]
    mult = jnp.dot(1.0 / (den + 1e-10), s_ref[...],
                   preferred_element_type=jnp.float32)
    o_ref[...] = tot * mult


def _sc_agg_body(wh_hbm, al_hbm, src3_hbm, dst3_hbm, onum_hbm, oden_hbm,
                 rows3, asrc3, adst3, dbuf2, sgidx, dgidx,
                 numer_s, denom_s, gsem, ssem, isem):
    n = wh_hbm.shape[0]
    ngrp = src3_hbm.shape[1]
    nchunk = ngrp * G
    c = lax.axis_index("c")
    s = lax.axis_index("s")
    w = c * NS + s

    rpt = n // NS
    row0 = s * rpt
    nfull = rpt // C
    rem = rpt - nfull * C

    zeros16 = jnp.zeros((LANES,), jnp.float32)

    def _zrows(i, _):
        r = i // (HF // LANES)
        j = i % (HF // LANES)
        rows3[0, r, pl.ds(j * LANES, LANES)] = zeros16
        return 0

    lax.fori_loop(0, C * (HF // LANES), _zrows, 0)

    lanes0 = lax.iota(jnp.int32, LANES)
    zrow = lax.shift_right_logical(lanes0, 3)
    zcol = lanes0 & 7

    for b in range(2):
        def _zdbuf(i, _):
            plsc.store_scatter(dbuf2.at[b], [zrow + i * 2, zcol], zeros16)
            return 0

        lax.fori_loop(0, C * DP // LANES, _zdbuf, 0)

    def _zcopy(j, _):
        pltpu.sync_copy(rows3.at[0], numer_s.at[pl.ds(row0 + j * C, C)])
        pltpu.sync_copy(dbuf2.at[0], denom_s.at[pl.ds(row0 + j * C, C)])
        return 0

    lax.fori_loop(0, nfull, _zcopy, 0)
    pltpu.sync_copy(rows3.at[0, pl.ds(0, rem)],
                    numer_s.at[pl.ds(row0 + nfull * C, rem)])
    pltpu.sync_copy(dbuf2.at[0, pl.ds(0, rem)],
                    denom_s.at[pl.ds(row0 + nfull * C, rem)])

    plsc.subcore_barrier()

    lanes = lax.iota(jnp.int32, LANES)

    def _sidx(t):
        g = t // G
        p = t - g * G
        return sgidx.at[g % 2, pl.ds(p * C, C)], dgidx.at[g % 2, pl.ds(p * C, C)]

    def _gather_descs(t):
        b = t % 3
        ssl, dsl = _sidx(t)
        return (
            pltpu.make_async_copy(wh_hbm.at[ssl], rows3.at[b], gsem.at[b]),
            pltpu.make_async_copy(al_hbm.at[ssl], asrc3.at[b], gsem.at[b]),
            pltpu.make_async_copy(al_hbm.at[dsl], adst3.at[b], gsem.at[b]),
        )

    def _scatter_descs(t):
        b = t % 2
        _, dsl = _sidx(t)
        return (
            pltpu.make_async_copy(rows3.at[t % 3], numer_s.at[dsl], ssem.at[b]),
            pltpu.make_async_copy(dbuf2.at[b], denom_s.at[dsl], ssem.at[b]),
        )

    def _idx_descs(q):
        qb = q % 2
        return (
            pltpu.make_async_copy(src3_hbm.at[w, q], sgidx.at[qb], isem.at[qb]),
            pltpu.make_async_copy(dst3_hbm.at[w, q], dgidx.at[qb], isem.at[qb]),
        )

    pltpu.sync_copy(src3_hbm.at[w, 0], sgidx.at[0])
    pltpu.sync_copy(dst3_hbm.at[w, 0], dgidx.at[0])
    for d in _idx_descs(1):
        d.start()
    for d in _gather_descs(0):
        d.start()
    for d in _gather_descs(1):
        d.start()

    def _chunk_body(t, _):
        for d in _gather_descs(t):
            d.wait()

        rows_r = rows3.at[t % 3]
        asrc_r = asrc3.at[t % 3]
        adst_r = adst3.at[t % 3]
        dbuf_r = dbuf2.at[t % 2]

        def _group_body(g, _):
            erow = g * LANES + lanes
            exs = []
            for h in range(N_HEADS):
                ch = jnp.full((LANES,), h, jnp.int32)
                ea = plsc.load_gather(asrc_r, [erow, ch])
                eb = plsc.load_gather(adst_r, [erow, ch + N_HEADS])
                ev = ea + eb
                ev = jnp.where(ev > 0, ev, ev * 0.2)
                ex = jnp.exp(ev)
                plsc.store_scatter(dbuf_r, [erow, ch], ex)
                exs.append(ex)
            for h in range(N_HEADS):
                for f in range(OUT_F):
                    col = jnp.full((LANES,), h * OUT_F + f, jnp.int32)
                    v = plsc.load_gather(rows_r, [erow, col])
                    plsc.store_scatter(rows_r, [erow, col], v * exs[h])
            return 0

        lax.fori_loop(0, C // LANES, _group_body, 0)

        @pl.when(t >= 1)
        def _():
            for d in _scatter_descs(t - 1):
                d.wait()

        for d in _scatter_descs(t):
            d.start(add=True)

        t2 = t + 2

        @pl.when(t2 < nchunk)
        def _():
            @pl.when(t2 % G == 0)
            def _():
                for d in _idx_descs(t2 // G):
                    d.wait()
            for d in _gather_descs(t2):
                d.start()

        @pl.when((t % G == 0) & (t >= 1))
        def _():
            q1 = t // G + 1

            @pl.when(q1 < ngrp)
            def _():
                for d in _idx_descs(q1):
                    d.start()

        return 0

    lax.fori_loop(0, nchunk, _chunk_body, 0)

    for d in _scatter_descs(nchunk - 1):
        d.wait()

    plsc.subcore_barrier()

    def _ocopy(j, _):
        r0 = row0 + j * C
        pltpu.sync_copy(numer_s.at[pl.ds(r0, C)], rows3.at[0])
        pltpu.sync_copy(rows3.at[0], onum_hbm.at[c, pl.ds(r0, C)])
        pltpu.sync_copy(denom_s.at[pl.ds(r0, C)], dbuf2.at[0])
        pltpu.sync_copy(dbuf2.at[0], oden_hbm.at[c, pl.ds(r0, C)])
        return 0

    lax.fori_loop(0, nfull, _ocopy, 0)
    r0 = row0 + nfull * C
    pltpu.sync_copy(numer_s.at[pl.ds(r0, rem)], rows3.at[0, pl.ds(0, rem)])
    pltpu.sync_copy(rows3.at[0, pl.ds(0, rem)], onum_hbm.at[c, pl.ds(r0, rem)])
    pltpu.sync_copy(denom_s.at[pl.ds(r0, rem)], dbuf2.at[0, pl.ds(0, rem)])
    pltpu.sync_copy(dbuf2.at[0, pl.ds(0, rem)], oden_hbm.at[c, pl.ds(r0, rem)])


def kernel(x, edge_index, W, a):
    n, in_f = x.shape
    e = edge_index.shape[1]
    epw = e // NW
    ngrp = epw // (G * C)
    src3 = edge_index[0].reshape(NW, ngrp, G * C)
    dst3 = edge_index[1].reshape(NW, ngrp, G * C)
    wt = W.T

    onehot = jnp.eye(N_HEADS, dtype=x.dtype)
    a_src = a[:, :OUT_F]
    a_dst = a[:, OUT_F:]
    aa = jnp.concatenate(
        [a_src[:, :, None] * onehot[:, None, :],
         a_dst[:, :, None] * onehot[:, None, :]], axis=-1).reshape(HF, 2 * N_HEADS)

    sel = jnp.kron(jnp.eye(N_HEADS, dtype=x.dtype), jnp.ones((1, OUT_F), x.dtype))

    bn = 1000
    wh, al = pl.pallas_call(
        _tc_prep_body,
        grid=(n // bn,),
        in_specs=[
            pl.BlockSpec((bn, in_f), lambda i: (i, 0)),
            pl.BlockSpec((in_f, HF), lambda i: (0, 0)),
            pl.BlockSpec((HF, 2 * N_HEADS), lambda i: (0, 0)),
        ],
        out_specs=[
            pl.BlockSpec((bn, HF), lambda i: (i, 0)),
            pl.BlockSpec((bn, 2 * N_HEADS), lambda i: (i, 0)),
        ],
        out_shape=[
            jax.ShapeDtypeStruct((n, HF), jnp.float32),
            jax.ShapeDtypeStruct((n, 2 * N_HEADS), jnp.float32),
        ],
    )(x, wt, aa)

    mesh = plsc.VectorSubcoreMesh(core_axis_name="c", subcore_axis_name="s",
                                  num_cores=NC, num_subcores=NS)
    sc_agg = pl.kernel(
        _sc_agg_body,
        out_type=[
            jax.ShapeDtypeStruct((NC, n, HF), jnp.float32),
            jax.ShapeDtypeStruct((NC, n, DP), jnp.float32),
        ],
        mesh=mesh,
        compiler_params=pltpu.CompilerParams(use_tc_tiling_on_sc=False,
                                             needs_layout_passes=False),
        scratch_types=[
            pltpu.VMEM((3, C, HF), jnp.float32),
            pltpu.VMEM((3, C, 2 * N_HEADS), jnp.float32),
            pltpu.VMEM((3, C, 2 * N_HEADS), jnp.float32),
            pltpu.VMEM((2, C, DP), jnp.float32),
            pltpu.VMEM((2, G * C), jnp.int32),
            pltpu.VMEM((2, G * C), jnp.int32),
            pltpu.VMEM_SHARED((n, HF), jnp.float32),
            pltpu.VMEM_SHARED((n, DP), jnp.float32),
            pltpu.SemaphoreType.DMA((3,)),
            pltpu.SemaphoreType.DMA((2,)),
            pltpu.SemaphoreType.DMA((2,)),
        ],
    )
    pnum, pden = sc_agg(wh, al, src3, dst3)

    out = pl.pallas_call(
        _tc_finish_body,
        grid=(n // bn,),
        in_specs=[
            pl.BlockSpec((NC, bn, HF), lambda i: (0, i, 0)),
            pl.BlockSpec((NC, bn, DP), lambda i: (0, i, 0)),
            pl.BlockSpec((N_HEADS, HF), lambda i: (0, 0)),
        ],
        out_specs=pl.BlockSpec((bn, HF), lambda i: (i, 0)),
        out_shape=jax.ShapeDtypeStruct((n, HF), jnp.float32),
    )(pnum, pden, sel)
    return out
